# baseline (speedup 1.0000x reference)
import functools

import jax
import jax.numpy as jnp
from jax import lax
from jax.experimental import pallas as pl
from jax.experimental.pallas import tpu as pltpu
from jax.experimental.pallas import tpu_sc as plsc

WH = 24
WW = 24
AREA_P1 = WH * WW + 1
N = AREA_P1 * AREA_P1
NUM_HEADS = 16
TABLE_ROWS = (2 * WH - 1) * (2 * WW - 1) + 3

_INFO = plsc.get_sparse_core_info()
NC = _INFO.num_cores
NS = _INFO.num_subcores
NW = NC * NS
LANES = _INFO.num_lanes

RS = 8
HT = 8
NSTRIPE = AREA_P1 // RS
NUNIT = NSTRIPE * 2
KMAX = 5
MV = 36
CTAIL = AREA_P1 - LANES


def _sc_bias_gather(table, idx_pad):
    mesh = plsc.VectorSubcoreMesh(core_axis_name="c", subcore_axis_name="s")

    @functools.partial(
        pl.kernel,
        mesh=mesh,
        out_type=jax.ShapeDtypeStruct(
            (AREA_P1, NUM_HEADS, AREA_P1), jnp.float32
        ),
        compiler_params=pltpu.CompilerParams(needs_layout_passes=False),
        scratch_types=[
            pltpu.VMEM((TABLE_ROWS * (NUM_HEADS + 1),), jnp.float32),
            pltpu.VMEM((RS, AREA_P1), jnp.int32),
            pltpu.VMEM((RS, AREA_P1), jnp.int32),
            pltpu.VMEM((RS, HT, AREA_P1), jnp.float32),
            pltpu.VMEM((RS, HT, AREA_P1), jnp.float32),
            pltpu.SemaphoreType.DMA,
            pltpu.SemaphoreType.DMA,
            pltpu.SemaphoreType.DMA,
            pltpu.SemaphoreType.DMA,
        ],
    )
    def k(table_hbm, idx_hbm, out_hbm, table_v, win_a, win_b,
          obuf_a, obuf_b, semo_a, semo_b, semw_a, semw_b):
        wid = lax.axis_index("s") * NC + lax.axis_index("c")
        pltpu.sync_copy(table_hbm, table_v)

        wins = (win_a, win_b)
        semws = (semw_a, semw_b)
        obufs = (obuf_a, obuf_b)
        semos = (semo_a, semo_b)

        def unit_params(s):
            it = lax.shift_right_logical(s, 1)
            u = jnp.bitwise_and(s, 1)
            return it, u

        def issue_win(s, p):
            it, _ = unit_params(s)
            base = pl.multiple_of(it * RS, 8)
            pltpu.async_copy(
                idx_hbm.at[pl.ds(base, RS), pl.ds(0, AREA_P1)],
                wins[p],
                semws[p],
            )

        def wait_win(p):
            pltpu.make_async_copy(
                idx_hbm.at[pl.ds(0, RS), pl.ds(0, AREA_P1)],
                wins[p],
                semws[p],
            ).wait()

        def drain_out(p):
            pltpu.make_async_copy(
                out_hbm.at[pl.ds(0, RS), pl.ds(0, HT), pl.ds(0, AREA_P1)],
                obufs[p],
                semos[p],
            ).wait()

        def fill_unit(s, p):
            _, u = unit_params(s)
            hbase = u * HT
            win = wins[p]
            obuf = obufs[p]
            iota = lax.broadcasted_iota(jnp.int32, (LANES,), 0)
            for r in range(RS):
                rv = jnp.full((LANES,), r, jnp.int32)

                def one_vec(c, rv=rv, r=r):
                    idxv = win[r, pl.ds(c, LANES)]
                    base = idxv * (NUM_HEADS + 1) + hbase
                    ci = c + iota
                    for h in range(HT):
                        g = plsc.load_gather(table_v, [base + h])
                        plsc.store_scatter(
                            obuf,
                            [rv, jnp.full((LANES,), h, jnp.int32), ci],
                            g,
                        )

                @plsc.parallel_loop(0, MV, 1, unroll=2)
                def _(m):
                    one_vec(m * LANES)

                one_vec(CTAIL)

        def send_unit(s, p):
            it, u = unit_params(s)
            pltpu.async_copy(
                obufs[p],
                out_hbm.at[
                    pl.ds(it * RS, RS),
                    pl.ds(pl.multiple_of(u * HT, 8), HT),
                    pl.ds(0, AREA_P1),
                ],
                semos[p],
            )

        issue_win(wid, 0)
        for kk in range(KMAX):
            p = kk & 1
            s = wid + NW * kk
            alive = s < NUNIT
            nxt = s + NW

            @pl.when(nxt < NUNIT)
            def _():
                issue_win(nxt, 1 - p)

            @pl.when(alive)
            def _():
                wait_win(p)
                if kk >= 2:
                    drain_out(p)
                fill_unit(s, p)
                send_unit(s, p)

        @pl.when(wid < 2)
        def _():
            u = wid
            pltpu.sync_copy(
                idx_hbm.at[pl.ds(NSTRIPE * RS, 1), pl.ds(0, AREA_P1)],
                win_a.at[pl.ds(0, 1)],
            )
            drain_out(0)
            drain_out(1)

            def one_vec_t(c):
                idxv = win_a[0, pl.ds(c, LANES)]
                base_v = idxv * (NUM_HEADS + 1) + u * HT
                for h in range(HT):
                    g = plsc.load_gather(table_v, [base_v + h])
                    obuf_a[0, h, pl.ds(c, LANES)] = g

            @plsc.parallel_loop(0, MV, 1, unroll=2)
            def _(m):
                one_vec_t(m * LANES)

            one_vec_t(CTAIL)
            pltpu.sync_copy(
                obuf_a.at[pl.ds(0, 1)],
                out_hbm.at[
                    pl.ds(NSTRIPE * RS, 1),
                    pl.ds(pl.multiple_of(u * HT, 8), HT),
                    pl.ds(0, AREA_P1),
                ],
            )

        @pl.when(wid >= 2)
        def _():
            drain_out(0)
            drain_out(1)

    return k(table, idx_pad)


def kernel(relative_position_bias_table, relative_position_index):
    table = jnp.pad(
        relative_position_bias_table.astype(jnp.float32), ((0, 0), (0, 1))
    ).reshape(-1)
    idx = relative_position_index.astype(jnp.int32)
    out = _sc_bias_gather(table, idx)
    return jnp.transpose(out, (1, 0, 2))


# speedup vs baseline: 1.0140x; 1.0140x over previous
import functools

import jax
import jax.numpy as jnp
from jax import lax
from jax.experimental import pallas as pl
from jax.experimental.pallas import tpu as pltpu
from jax.experimental.pallas import tpu_sc as plsc

WH = 24
WW = 24
AREA_P1 = WH * WW + 1
N = AREA_P1 * AREA_P1
NUM_HEADS = 16
TABLE_ROWS = (2 * WH - 1) * (2 * WW - 1) + 3

_INFO = plsc.get_sparse_core_info()
NC = _INFO.num_cores
NS = _INFO.num_subcores
NW = NC * NS
LANES = _INFO.num_lanes

RS = 8
HT = 8
NSTRIPE = AREA_P1 // RS
NUNIT = NSTRIPE * 2
KMAX = 5
WIN = RS * AREA_P1 + 16
MV = 36
CTAIL = AREA_P1 - LANES
NIDX = N + 4096


def _sc_bias_gather(table, idx_pad):
    mesh = plsc.VectorSubcoreMesh(core_axis_name="c", subcore_axis_name="s")

    @functools.partial(
        pl.kernel,
        mesh=mesh,
        out_type=jax.ShapeDtypeStruct(
            (AREA_P1, NUM_HEADS, AREA_P1), jnp.float32
        ),
        compiler_params=pltpu.CompilerParams(needs_layout_passes=False),
        scratch_types=[
            pltpu.VMEM((TABLE_ROWS * (NUM_HEADS + 1),), jnp.float32),
            pltpu.VMEM((WIN,), jnp.int32),
            pltpu.VMEM((WIN,), jnp.int32),
            pltpu.VMEM((RS, HT, AREA_P1), jnp.float32),
            pltpu.VMEM((RS, HT, AREA_P1), jnp.float32),
            pltpu.SemaphoreType.DMA,
            pltpu.SemaphoreType.DMA,
            pltpu.SemaphoreType.DMA,
            pltpu.SemaphoreType.DMA,
        ],
    )
    def k(table_hbm, idx_hbm, out_hbm, table_v, win_a, win_b,
          obuf_a, obuf_b, semo_a, semo_b, semw_a, semw_b):
        wid = lax.axis_index("s") * NC + lax.axis_index("c")
        pltpu.sync_copy(table_hbm, table_v)

        wins = (win_a, win_b)
        semws = (semw_a, semw_b)
        obufs = (obuf_a, obuf_b)
        semos = (semo_a, semo_b)

        def unit_params(s):
            it = lax.shift_right_logical(s, 1)
            u = jnp.bitwise_and(s, 1)
            return it, u

        def issue_win(s, p):
            it, _ = unit_params(s)
            base = pl.multiple_of(it * (RS * AREA_P1), 8)
            pltpu.async_copy(
                idx_hbm.at[pl.ds(base, WIN)], wins[p], semws[p]
            )

        def wait_win(p):
            pltpu.make_async_copy(
                idx_hbm.at[pl.ds(0, WIN)], wins[p], semws[p]
            ).wait()

        def drain_out(p):
            pltpu.make_async_copy(
                out_hbm.at[pl.ds(0, RS), pl.ds(0, HT), pl.ds(0, AREA_P1)],
                obufs[p],
                semos[p],
            ).wait()

        def fill_unit(s, p):
            _, u = unit_params(s)
            hbase = u * HT
            win = wins[p]
            obuf = obufs[p]
            iota = lax.broadcasted_iota(jnp.int32, (LANES,), 0)
            for r in range(RS):
                rv = jnp.full((LANES,), r, jnp.int32)

                def one_vec(c, rv=rv, r=r):
                    idxv = win[pl.ds(r * AREA_P1 + c, LANES)]
                    base = idxv * (NUM_HEADS + 1) + hbase
                    ci = c + iota
                    for h in range(HT):
                        g = plsc.load_gather(table_v, [base + h])
                        plsc.store_scatter(
                            obuf,
                            [rv, jnp.full((LANES,), h, jnp.int32), ci],
                            g,
                        )

                @plsc.parallel_loop(0, MV, 1, unroll=2)
                def _(m):
                    one_vec(m * LANES)

                one_vec(CTAIL)

        def send_unit(s, p):
            it, u = unit_params(s)
            pltpu.async_copy(
                obufs[p],
                out_hbm.at[
                    pl.ds(it * RS, RS),
                    pl.ds(pl.multiple_of(u * HT, 8), HT),
                    pl.ds(0, AREA_P1),
                ],
                semos[p],
            )

        issue_win(wid, 0)
        for kk in range(KMAX):
            p = kk & 1
            s = wid + NW * kk
            alive = s < NUNIT
            nxt = s + NW

            @pl.when(nxt < NUNIT)
            def _():
                issue_win(nxt, 1 - p)

            @pl.when(alive)
            def _():
                wait_win(p)
                if kk >= 2:
                    drain_out(p)
                fill_unit(s, p)
                send_unit(s, p)

        @pl.when(wid < 2)
        def _():
            u = wid
            base = pl.multiple_of(NSTRIPE * (RS * AREA_P1), 8)
            pltpu.sync_copy(idx_hbm.at[pl.ds(base, WIN)], win_a)
            drain_out(0)
            drain_out(1)

            def one_vec_t(c):
                idxv = win_a[pl.ds(c, LANES)]
                base_v = idxv * (NUM_HEADS + 1) + u * HT
                for h in range(HT):
                    g = plsc.load_gather(table_v, [base_v + h])
                    obuf_a[0, h, pl.ds(c, LANES)] = g

            @plsc.parallel_loop(0, MV, 1, unroll=2)
            def _(m):
                one_vec_t(m * LANES)

            one_vec_t(CTAIL)
            pltpu.sync_copy(
                obuf_a.at[pl.ds(0, 1)],
                out_hbm.at[
                    pl.ds(NSTRIPE * RS, 1),
                    pl.ds(pl.multiple_of(u * HT, 8), HT),
                    pl.ds(0, AREA_P1),
                ],
            )

        @pl.when(wid >= 2)
        def _():
            drain_out(0)
            drain_out(1)

    return k(table, idx_pad)


def kernel(relative_position_bias_table, relative_position_index):
    table = jnp.pad(
        relative_position_bias_table.astype(jnp.float32), ((0, 0), (0, 1))
    ).reshape(-1)
    idx = relative_position_index.reshape(-1).astype(jnp.int32)
    idx_pad = jnp.concatenate([idx, jnp.zeros((NIDX - N,), jnp.int32)])
    out = _sc_bias_gather(table, idx_pad)
    return jnp.transpose(out, (1, 0, 2))


# speedup vs baseline: 1.0565x; 1.0419x over previous
import functools

import jax
import jax.numpy as jnp
from jax import lax
from jax.experimental import pallas as pl
from jax.experimental.pallas import tpu as pltpu
from jax.experimental.pallas import tpu_sc as plsc

WH = 24
WW = 24
AREA_P1 = WH * WW + 1
N = AREA_P1 * AREA_P1
NUM_HEADS = 16
TABLE_ROWS = (2 * WH - 1) * (2 * WW - 1) + 3

_INFO = plsc.get_sparse_core_info()
NC = _INFO.num_cores
NS = _INFO.num_subcores
NW = NC * NS
LANES = _INFO.num_lanes

RS = 4
HT = 8
NSTRIPE = AREA_P1 // RS
NUNIT = NSTRIPE * 2
KMAX = NUNIT // NW
WIN = RS * AREA_P1 + 4
MV = 36
CTAIL = AREA_P1 - LANES
NIDX = N + 4096


def _sc_bias_gather(table, idx_pad):
    mesh = plsc.VectorSubcoreMesh(core_axis_name="c", subcore_axis_name="s")

    @functools.partial(
        pl.kernel,
        mesh=mesh,
        out_type=jax.ShapeDtypeStruct(
            (AREA_P1, NUM_HEADS, AREA_P1), jnp.float32
        ),
        compiler_params=pltpu.CompilerParams(needs_layout_passes=False),
        scratch_types=[
            pltpu.VMEM((TABLE_ROWS * (NUM_HEADS + 1),), jnp.float32),
            pltpu.VMEM((WIN,), jnp.int32),
            pltpu.VMEM((WIN,), jnp.int32),
            pltpu.VMEM((RS, HT, AREA_P1), jnp.float32),
            pltpu.VMEM((RS, HT, AREA_P1), jnp.float32),
            pltpu.SemaphoreType.DMA,
            pltpu.SemaphoreType.DMA,
            pltpu.SemaphoreType.DMA,
            pltpu.SemaphoreType.DMA,
        ],
    )
    def k(table_hbm, idx_hbm, out_hbm, table_v, win_a, win_b,
          obuf_a, obuf_b, semo_a, semo_b, semw_a, semw_b):
        wid = lax.axis_index("s") * NC + lax.axis_index("c")
        pltpu.sync_copy(table_hbm, table_v)

        wins = (win_a, win_b)
        semws = (semw_a, semw_b)
        obufs = (obuf_a, obuf_b)
        semos = (semo_a, semo_b)

        def unit_params(s):
            it = lax.shift_right_logical(s, 1)
            u = jnp.bitwise_and(s, 1)
            return it, u

        def issue_win(s, p):
            it, _ = unit_params(s)
            base = it * (RS * AREA_P1)
            d = jnp.bitwise_and(base, 7)
            s8 = pl.multiple_of(base - d, 8)
            pltpu.async_copy(
                idx_hbm.at[pl.ds(s8, WIN)], wins[p], semws[p]
            )

        def wait_win(p):
            pltpu.make_async_copy(
                idx_hbm.at[pl.ds(0, WIN)], wins[p], semws[p]
            ).wait()

        def drain_out(p):
            pltpu.make_async_copy(
                out_hbm.at[pl.ds(0, RS), pl.ds(0, HT), pl.ds(0, AREA_P1)],
                obufs[p],
                semos[p],
            ).wait()

        def fill_unit(s, p):
            it, u = unit_params(s)
            d = jnp.bitwise_and(it * (RS * AREA_P1), 7)
            hbase = u * HT
            win = wins[p]
            obuf = obufs[p]
            iota = lax.broadcasted_iota(jnp.int32, (LANES,), 0)
            for r in range(RS):
                rv = jnp.full((LANES,), r, jnp.int32)

                def one_vec(c, rv=rv, r=r):
                    idxv = win[pl.ds(d + r * AREA_P1 + c, LANES)]
                    base = idxv * (NUM_HEADS + 1) + hbase
                    ci = c + iota
                    for h in range(HT):
                        g = plsc.load_gather(table_v, [base + h])
                        plsc.store_scatter(
                            obuf,
                            [rv, jnp.full((LANES,), h, jnp.int32), ci],
                            g,
                        )

                @plsc.parallel_loop(0, MV, 1, unroll=2)
                def _(m):
                    one_vec(m * LANES)

                one_vec(CTAIL)

        def send_unit(s, p):
            it, u = unit_params(s)
            pltpu.async_copy(
                obufs[p],
                out_hbm.at[
                    pl.ds(it * RS, RS),
                    pl.ds(pl.multiple_of(u * HT, 8), HT),
                    pl.ds(0, AREA_P1),
                ],
                semos[p],
            )

        issue_win(wid, 0)
        for kk in range(KMAX):
            p = kk & 1
            s = wid + NW * kk
            alive = s < NUNIT
            nxt = s + NW

            @pl.when(nxt < NUNIT)
            def _():
                issue_win(nxt, 1 - p)

            @pl.when(alive)
            def _():
                wait_win(p)
                if kk >= 2:
                    drain_out(p)
                fill_unit(s, p)
                send_unit(s, p)

        @pl.when(wid < 2)
        def _():
            u = wid
            base = pl.multiple_of(NSTRIPE * (RS * AREA_P1), 8)
            pltpu.sync_copy(idx_hbm.at[pl.ds(base, WIN)], win_a)
            drain_out(0)
            drain_out(1)

            def one_vec_t(c):
                idxv = win_a[pl.ds(c, LANES)]
                base_v = idxv * (NUM_HEADS + 1) + u * HT
                for h in range(HT):
                    g = plsc.load_gather(table_v, [base_v + h])
                    obuf_a[0, h, pl.ds(c, LANES)] = g

            @plsc.parallel_loop(0, MV, 1, unroll=2)
            def _(m):
                one_vec_t(m * LANES)

            one_vec_t(CTAIL)
            pltpu.sync_copy(
                obuf_a.at[pl.ds(0, 1)],
                out_hbm.at[
                    pl.ds(NSTRIPE * RS, 1),
                    pl.ds(pl.multiple_of(u * HT, 8), HT),
                    pl.ds(0, AREA_P1),
                ],
            )

        @pl.when(wid >= 2)
        def _():
            drain_out(0)
            drain_out(1)

    return k(table, idx_pad)


def kernel(relative_position_bias_table, relative_position_index):
    table = jnp.pad(
        relative_position_bias_table.astype(jnp.float32), ((0, 0), (0, 1))
    ).reshape(-1)
    idx = relative_position_index.reshape(-1).astype(jnp.int32)
    idx_pad = jnp.concatenate([idx, jnp.zeros((NIDX - N,), jnp.int32)])
    out = _sc_bias_gather(table, idx_pad)
    return jnp.transpose(out, (1, 0, 2))
